# baseline (speedup 1.0000x reference)
import jax
import jax.numpy as jnp
from jax.experimental import pallas as pl
from jax.experimental.pallas import tpu as pltpu

_N = 1024
_D = 64
_BM = 4096


def _vq_block(e_ref, w_ref, idx_ref, q_ref):
    e = e_ref[...]
    w = w_ref[...]
    dot = jax.lax.dot_general(e, w, (((1,), (1,)), ((), ())),
                              preferred_element_type=jnp.float32)
    e_sq = jnp.sum(e * e, axis=1, keepdims=True)
    w_sq = jnp.sum(w * w, axis=1)[None, :]
    dist = e_sq + w_sq - 2.0 * dot
    m = jnp.min(dist, axis=1, keepdims=True)
    iota = jax.lax.broadcasted_iota(jnp.int32, dist.shape, 1)
    idx = jnp.min(jnp.where(dist == m, iota, _N), axis=1)
    idx_ref[...] = idx[None, :]
    onehot = (iota == idx[:, None]).astype(jnp.float32)
    q = jax.lax.dot_general(onehot, w, (((1,), (0,)), ((), ())),
                            preferred_element_type=jnp.float32)
    q_ref[...] = q


def kernel(x, W):
    perm = (0,) + tuple(range(2, x.ndim)) + (1,)
    encoded_permuted = jnp.transpose(x, perm)
    permuted_shape = encoded_permuted.shape
    encoded_flat = encoded_permuted.reshape(-1, permuted_shape[-1])
    M = encoded_flat.shape[0]

    idx2, q = pl.pallas_call(
        _vq_block,
        grid=(M // _BM,),
        in_specs=[
            pl.BlockSpec((_BM, _D), lambda i: (i, 0)),
            pl.BlockSpec((_N, _D), lambda i: (0, 0)),
        ],
        out_specs=[
            pl.BlockSpec((1, _BM), lambda i: (0, i)),
            pl.BlockSpec((_BM, _D), lambda i: (i, 0)),
        ],
        out_shape=[
            jax.ShapeDtypeStruct((1, M), jnp.int32),
            jax.ShapeDtypeStruct((M, _D), jnp.float32),
        ],
        compiler_params=pltpu.CompilerParams(
            dimension_semantics=("arbitrary",),
        ),
    )(encoded_flat, W)

    codebook_indices = idx2.reshape(M)
    quantized_flat = q
    num_dims = len(permuted_shape)
    quantized_permuted = quantized_flat.reshape(permuted_shape)
    old_dims = (0,) + (num_dims - 1,) + tuple(range(1, num_dims - 1))
    quantized = jnp.transpose(quantized_permuted, old_dims)
    return (encoded_flat, quantized_flat, codebook_indices, quantized)
